# baseline (speedup 1.0000x reference)
import functools

import jax
import jax.numpy as jnp
from jax import lax
from jax.experimental import pallas as pl
from jax.experimental.pallas import tpu as pltpu
from jax.experimental.pallas import tpu_sc as plsc

NUM_ENT = 1000000
EMBED_DIM = 64
BATCH = 4096
PAIRS = 50

_NC = 2
_NS = 16
_NW = _NC * _NS

_TOTAL = BATCH * PAIRS * 2
_PER_W = _TOTAL // _NW
_CHUNK = 128
_NCHUNK = _PER_W // _CHUNK


def _sc_gather(table, idx3):
    mesh = plsc.VectorSubcoreMesh(core_axis_name="c", subcore_axis_name="s")

    @functools.partial(
        pl.kernel,
        mesh=mesh,
        out_type=jax.ShapeDtypeStruct((_TOTAL, EMBED_DIM), jnp.float32),
        scratch_types=[
            pltpu.VMEM((_NCHUNK, _CHUNK), jnp.int32),
            pltpu.VMEM((_CHUNK, EMBED_DIM), jnp.float32),
            pltpu.SemaphoreType.DMA,
        ],
        compiler_params=pltpu.CompilerParams(use_tc_tiling_on_sc=False),
    )
    def k(table_hbm, idx_hbm, out_hbm, idx_v, rows_v, gsem):
        wid = lax.axis_index("s") * _NC + lax.axis_index("c")
        base = wid * _PER_W
        pltpu.sync_copy(idx_hbm.at[wid], idx_v)

        def chunk(j, carry):
            pltpu.async_copy(table_hbm.at[idx_v.at[j]], rows_v, gsem).wait()
            pltpu.sync_copy(rows_v, out_hbm.at[pl.ds(base + j * _CHUNK, _CHUNK)])
            return carry

        lax.fori_loop(0, _NCHUNK, chunk, 0)

    return k(table, idx3)


def kernel(idx, embedding_weight):
    idx3 = idx.reshape(_NW, _NCHUNK, _CHUNK)
    out = _sc_gather(embedding_weight, idx3)
    return out.reshape(BATCH, PAIRS, 2, EMBED_DIM)


# speedup vs baseline: 1.0557x; 1.0557x over previous
import functools

import jax
import jax.numpy as jnp
from jax import lax
from jax.experimental import pallas as pl
from jax.experimental.pallas import tpu as pltpu
from jax.experimental.pallas import tpu_sc as plsc

NUM_ENT = 1000000
EMBED_DIM = 64
BATCH = 4096
PAIRS = 50

_NC = 2
_NS = 16
_NW = _NC * _NS

_TOTAL = BATCH * PAIRS * 2
_PER_W = _TOTAL // _NW
_CHUNK = 128
_NCHUNK = _PER_W // _CHUNK
_NBUF = 10
_NOUTER = _NCHUNK // _NBUF


def _sc_gather(table, idx3):
    mesh = plsc.VectorSubcoreMesh(core_axis_name="c", subcore_axis_name="s")

    @functools.partial(
        pl.kernel,
        mesh=mesh,
        out_type=jax.ShapeDtypeStruct((_TOTAL, EMBED_DIM), jnp.float32),
        scratch_types=[
            pltpu.VMEM((_NCHUNK, _CHUNK), jnp.int32),
            pltpu.VMEM((_NBUF, _CHUNK, EMBED_DIM), jnp.float32),
            pltpu.SemaphoreType.DMA((_NBUF,)),
            pltpu.SemaphoreType.DMA((_NBUF,)),
        ],
        compiler_params=pltpu.CompilerParams(use_tc_tiling_on_sc=False),
    )
    def k(table_hbm, idx_hbm, out_hbm, idx_v, rows_v, gsem, osem):
        wid = lax.axis_index("s") * _NC + lax.axis_index("c")
        base = wid * _PER_W
        pltpu.sync_copy(idx_hbm.at[wid], idx_v)

        def gstart(b, j):
            pltpu.make_async_copy(
                table_hbm.at[idx_v.at[j]], rows_v.at[b], gsem.at[b]
            ).start()

        def gwait(b):
            pltpu.make_async_copy(
                table_hbm.at[idx_v.at[0]], rows_v.at[b], gsem.at[b]
            ).wait()

        def ostart(b, j):
            pltpu.make_async_copy(
                rows_v.at[b],
                out_hbm.at[pl.ds(base + j * _CHUNK, _CHUNK)],
                osem.at[b],
            ).start()

        def owait(b):
            pltpu.make_async_copy(
                rows_v.at[b], out_hbm.at[pl.ds(base, _CHUNK)], osem.at[b]
            ).wait()

        for b in range(_NBUF):
            gstart(b, b)

        def body(o, carry):
            for b in range(_NBUF):
                gwait(b)
                ostart(b, o * _NBUF + b)
            for b in range(_NBUF):
                owait(b)
                gstart(b, (o + 1) * _NBUF + b)
            return carry

        last = _NOUTER - 1
        lax.fori_loop(0, last, body, 0)
        for b in range(_NBUF):
            gwait(b)
            ostart(b, last * _NBUF + b)
        for b in range(_NBUF):
            owait(b)

    return k(table, idx3)


def kernel(idx, embedding_weight):
    idx3 = idx.reshape(_NW, _NCHUNK, _CHUNK)
    out = _sc_gather(embedding_weight, idx3)
    return out.reshape(BATCH, PAIRS, 2, EMBED_DIM)
